# baseline (speedup 1.0000x reference)
import functools

import jax
import jax.numpy as jnp
from jax import lax
from jax.experimental import pallas as pl
from jax.experimental.pallas import tpu as pltpu
from jax.experimental.pallas import tpu_sc as plsc

N_NODES = 10000
N_EDGES = 320000
D_FEAT = 128
D_HALF = D_FEAT // 2
D_EDGE = 4
D_EDGE_PAD = 16

NC = 2
NS = 16

ROWS_PAD = 10240
ROWS_PER_TILE = ROWS_PAD // NS
DUMMY_ROW = 10200
EDGES_PER_TILE = N_EDGES // NS
MEGA = 2000
N_MEGA = EDGES_PER_TILE // MEGA
SUB = 80
N_SUB = MEGA // SUB
VECS_PER_MEGA = MEGA // 16
RB = 5
EFF_PER_CHUNK = 7


def _make_sc_kernel():
    mesh = plsc.VectorSubcoreMesh(core_axis_name="c", subcore_axis_name="s")

    @functools.partial(
        pl.kernel,
        out_type=[
            jax.ShapeDtypeStruct((NC, ROWS_PAD, D_HALF), jnp.float32),
            jax.ShapeDtypeStruct((ROWS_PAD, D_EDGE_PAD), jnp.float32),
        ],
        mesh=mesh,
        compiler_params=pltpu.CompilerParams(
            needs_layout_passes=False, use_tc_tiling_on_sc=False),
        scratch_types=[
            pltpu.VMEM((N_NODES,), jnp.int32),
            pltpu.VMEM((2, MEGA), jnp.int32),
            pltpu.VMEM((2, MEGA), jnp.int32),
            pltpu.VMEM((RB, SUB, D_EDGE_PAD), jnp.float32),
            pltpu.VMEM((2, N_SUB, SUB), jnp.int32),
            pltpu.VMEM((RB, SUB, D_HALF), jnp.float32),
            pltpu.VMEM_SHARED((ROWS_PAD, D_HALF), jnp.float32),
            pltpu.VMEM_SHARED((ROWS_PAD, D_EDGE_PAD), jnp.float32),
            pltpu.SemaphoreType.DMA((3,)),
            pltpu.SemaphoreType.DMA((RB,)),
            pltpu.SemaphoreType.DMA((RB,)),
            pltpu.SemaphoreType.DMA((RB,)),
            pltpu.SemaphoreType.DMA((RB,)),
        ],
    )
    def sc_kernel(xh_hbm, ei_hbm, ea_hbm, mask_hbm, zg_hbm, ze_hbm,
                  g_out, e_out,
                  mask_v, srcb, dstb, ea16, effb, rows, g_sh, e_sh,
                  stsem, gsem, wsem, sgsem, sesem):
        cid = lax.axis_index("c")
        sid = lax.axis_index("s")
        r0 = sid * ROWS_PER_TILE
        is0 = cid == 0
        xcol = xh_hbm.at[cid]

        pltpu.sync_copy(zg_hbm.at[pl.ds(r0, ROWS_PER_TILE)],
                        g_sh.at[pl.ds(r0, ROWS_PER_TILE)])

        @pl.when(is0)
        def _():
            pltpu.sync_copy(ze_hbm.at[pl.ds(r0, ROWS_PER_TILE)],
                            e_sh.at[pl.ds(r0, ROWS_PER_TILE)])
        for j in range(RB):
            pltpu.sync_copy(ze_hbm.at[pl.ds(0, SUB)], ea16.at[j])
        pltpu.sync_copy(mask_hbm, mask_v)

        plsc.subcore_barrier()

        ebase = sid * EDGES_PER_TILE

        def stage_start(pp, mm):
            b = ebase + mm * MEGA
            return [
                pltpu.make_async_copy(ei_hbm.at[0, pl.ds(b, MEGA)],
                                      srcb.at[pp], stsem.at[0]),
                pltpu.make_async_copy(ei_hbm.at[1, pl.ds(b, MEGA)],
                                      dstb.at[pp], stsem.at[1]),
            ]

        def eff_iters(qq, lo, hi):
            def body(i, _):
                sv = srcb[qq, pl.ds(i * 16, 16)]
                dv = dstb[qq, pl.ds(i * 16, 16)]
                ms = plsc.load_gather(mask_v, [sv])
                md = plsc.load_gather(mask_v, [dv])
                keep = (ms & md) > 0
                effb[qq, i // 5, pl.ds((i % 5) * 16, 16)] = (
                    jnp.where(keep, dv, DUMMY_ROW))
                return 0
            lax.fori_loop(lo, hi, body, 0)

        for d in stage_start(0, 0):
            d.start()
            d.wait()
        eff_iters(0, 0, VECS_PER_MEGA)

        def mega_body(m, _):
            p = m % 2
            q = 1 - p
            m_next = jnp.minimum(m + 1, N_MEGA - 1)

            stage_descs = stage_start(q, m_next)
            for d in stage_descs:
                d.start()

            def gstart(j):
                d = pltpu.make_async_copy(
                    xcol.at[srcb.at[p, pl.ds(j * SUB, SUB)]],
                    rows.at[j % RB], gsem.at[j % RB])
                d.start()
                return d

            def wstart(j):
                d = pltpu.make_async_copy(
                    ea_hbm.at[pl.ds(ebase + m * MEGA + j * SUB, SUB), :],
                    ea16.at[j % RB, :, pl.ds(0, D_EDGE)], wsem.at[j % RB])
                d.start()
                return d

            gd = [None] * N_SUB
            wd = [None] * N_SUB
            sg = [None] * N_SUB
            se = [None] * N_SUB
            gd[0] = gstart(0)
            gd[1] = gstart(1)
            wd[0] = wstart(0)
            wd[1] = wstart(1)
            for k in range(N_SUB):
                if k >= 3:
                    sg[k - 3].wait()
                    se[k - 3].wait()
                if k + 2 < N_SUB:
                    gd[k + 2] = gstart(k + 2)
                    wd[k + 2] = wstart(k + 2)
                gd[k].wait()
                wd[k].wait()
                idx = effb.at[p, k]
                sg[k] = pltpu.make_async_copy(rows.at[k % RB],
                                              g_sh.at[idx], sgsem.at[k % RB])
                sg[k].start(add=True)
                se[k] = pltpu.make_async_copy(ea16.at[k % RB],
                                              e_sh.at[idx], sesem.at[k % RB])
                se[k].start(add=True)
                if k == 4:
                    for d in stage_descs:
                        d.wait()
                if k >= 5:
                    lo = (k - 5) * EFF_PER_CHUNK
                    hi = min(VECS_PER_MEGA, lo + EFF_PER_CHUNK)
                    if lo < hi:
                        eff_iters(q, lo, hi)
            for k in range(N_SUB - 3, N_SUB):
                sg[k].wait()
                se[k].wait()
            return 0

        lax.fori_loop(0, N_MEGA, mega_body, 0)

        plsc.subcore_barrier()

        pltpu.sync_copy(g_sh.at[pl.ds(r0, ROWS_PER_TILE)],
                        g_out.at[cid, pl.ds(r0, ROWS_PER_TILE)])

        @pl.when(is0)
        def _():
            pltpu.sync_copy(e_sh.at[pl.ds(r0, ROWS_PER_TILE)],
                            e_out.at[pl.ds(r0, ROWS_PER_TILE)])

    return sc_kernel


def _dense_body(x_ref, gp_ref, ep_ref, ws_ref, wn_ref, we_ref, b_ref, m_ref, o_ref):
    g = jnp.concatenate([gp_ref[0], gp_ref[1]], axis=-1)
    acc = jnp.dot(x_ref[...], ws_ref[...], preferred_element_type=jnp.float32)
    acc = acc + jnp.dot(g, wn_ref[...], preferred_element_type=jnp.float32)
    acc = acc + jnp.dot(ep_ref[...], we_ref[...], preferred_element_type=jnp.float32)
    acc = acc + b_ref[...]
    o_ref[...] = jnp.maximum(acc, 0.0) * m_ref[...]


_R = 400


def _dense_call(x, gp, ep, W_self, W_nbr, W_edge, b2, m2):
    return pl.pallas_call(
        _dense_body,
        grid=(N_NODES // _R,),
        in_specs=[
            pl.BlockSpec((_R, D_FEAT), lambda i: (i, 0)),
            pl.BlockSpec((NC, _R, D_HALF), lambda i: (0, i, 0)),
            pl.BlockSpec((_R, D_EDGE_PAD), lambda i: (i, 0)),
            pl.BlockSpec((D_FEAT, D_FEAT), lambda i: (0, 0)),
            pl.BlockSpec((D_FEAT, D_FEAT), lambda i: (0, 0)),
            pl.BlockSpec((D_EDGE_PAD, D_FEAT), lambda i: (0, 0)),
            pl.BlockSpec((1, D_FEAT), lambda i: (0, 0)),
            pl.BlockSpec((_R, 1), lambda i: (i, 0)),
        ],
        out_specs=pl.BlockSpec((_R, D_FEAT), lambda i: (i, 0)),
        out_shape=jax.ShapeDtypeStruct((N_NODES, D_FEAT), jnp.float32),
    )(x, gp, ep, W_self, W_nbr, W_edge, b2, m2)


def kernel(x, edge_attr, W_self, W_nbr, W_edge, b, edge_index, mask):
    mask_i32 = mask.astype(jnp.int32)
    zg = jnp.zeros((ROWS_PAD, D_HALF), jnp.float32)
    ze = jnp.zeros((ROWS_PAD, D_EDGE_PAD), jnp.float32)
    sc = _make_sc_kernel()
    xh = x.reshape(N_NODES, NC, D_HALF).swapaxes(0, 1)
    gp, ep = sc(xh, edge_index, edge_attr, mask_i32, zg, ze)
    b2 = b.reshape(1, D_FEAT)
    m2 = mask.astype(jnp.float32).reshape(N_NODES, 1)
    we_pad = jnp.pad(W_edge, ((0, D_EDGE_PAD - D_EDGE), (0, 0)))
    return _dense_call(x, gp, ep, W_self, W_nbr, we_pad, b2, m2)


# speedup vs baseline: 1.5977x; 1.5977x over previous
import functools

import jax
import jax.numpy as jnp
from jax import lax
from jax.experimental import pallas as pl
from jax.experimental.pallas import tpu as pltpu
from jax.experimental.pallas import tpu_sc as plsc

N_NODES = 10000
N_EDGES = 320000
D_FEAT = 128
D_EDGE = 4
D_EDGE_PAD = 16

NC = 2
NS = 16
NW = NC * NS

ROWS_PAD = 10240
ROWS_PER_TILE = ROWS_PAD // NS
DUMMY_ROW = 10200
EDGES_PER_WORKER = N_EDGES // NW
MEGA = 400
N_MEGA = EDGES_PER_WORKER // MEGA
SUB = 80
N_SUB = MEGA // SUB
VECS_PER_MEGA = MEGA // 16
RB = 2


def _make_sc_kernel():
    mesh = plsc.VectorSubcoreMesh(core_axis_name="c", subcore_axis_name="s")

    @functools.partial(
        pl.kernel,
        out_type=[
            jax.ShapeDtypeStruct((NC, ROWS_PAD, D_FEAT), jnp.float32),
            jax.ShapeDtypeStruct((NC, ROWS_PAD, D_EDGE_PAD), jnp.float32),
        ],
        mesh=mesh,
        compiler_params=pltpu.CompilerParams(
            needs_layout_passes=False, use_tc_tiling_on_sc=False),
        scratch_types=[
            pltpu.VMEM((N_NODES,), jnp.int32),
            pltpu.VMEM((2, MEGA), jnp.int32),
            pltpu.VMEM((2, MEGA), jnp.int32),
            pltpu.VMEM((RB, SUB * D_EDGE), jnp.float32),
            pltpu.VMEM((RB, SUB, D_EDGE_PAD), jnp.float32),
            pltpu.VMEM((2, N_SUB, SUB), jnp.int32),
            pltpu.VMEM((RB, SUB, D_FEAT), jnp.float32),
            pltpu.VMEM_SHARED((ROWS_PAD, D_FEAT), jnp.float32),
            pltpu.VMEM_SHARED((ROWS_PAD, D_EDGE_PAD), jnp.float32),
            pltpu.SemaphoreType.DMA((2,)),
            pltpu.SemaphoreType.DMA((RB,)),
            pltpu.SemaphoreType.DMA((RB,)),
            pltpu.SemaphoreType.DMA((RB,)),
            pltpu.SemaphoreType.DMA((RB,)),
        ],
    )
    def sc_kernel(x_hbm, ei_hbm, ea_hbm, mask_hbm, zg_hbm, ze_hbm,
                  g_out, e_out,
                  mask_v, srcb, dstb, ea4, ea16, effb, rows, g_sh, e_sh,
                  stsem, gsem, wsem, sgsem, sesem):
        cid = lax.axis_index("c")
        sid = lax.axis_index("s")
        wid = cid * NS + sid
        r0 = sid * ROWS_PER_TILE

        pltpu.sync_copy(zg_hbm.at[pl.ds(r0, ROWS_PER_TILE)],
                        g_sh.at[pl.ds(r0, ROWS_PER_TILE)])
        pltpu.sync_copy(ze_hbm.at[pl.ds(r0, ROWS_PER_TILE)],
                        e_sh.at[pl.ds(r0, ROWS_PER_TILE)])
        for j in range(RB):
            pltpu.sync_copy(ze_hbm.at[pl.ds(0, SUB)], ea16.at[j])
        pltpu.sync_copy(mask_hbm, mask_v)

        plsc.subcore_barrier()

        lane = lax.iota(jnp.int32, 16)
        row_off = lane // D_EDGE
        col_off = lane % D_EDGE

        ebase = wid * EDGES_PER_WORKER

        def stage_start(pp, mm):
            b = ebase + mm * MEGA
            return [
                pltpu.make_async_copy(ei_hbm.at[0, pl.ds(b, MEGA)],
                                      srcb.at[pp], stsem.at[0]),
                pltpu.make_async_copy(ei_hbm.at[1, pl.ds(b, MEGA)],
                                      dstb.at[pp], stsem.at[1]),
            ]

        def eff_iters(qq, lo, hi):
            def body(i, _):
                sv = srcb[qq, pl.ds(i * 16, 16)]
                dv = dstb[qq, pl.ds(i * 16, 16)]
                ms = plsc.load_gather(mask_v, [sv])
                md = plsc.load_gather(mask_v, [dv])
                keep = (ms & md) > 0
                effb[qq, i // 5, pl.ds((i % 5) * 16, 16)] = (
                    jnp.where(keep, dv, DUMMY_ROW))
                return 0
            lax.fori_loop(lo, hi, body, 0)

        for d in stage_start(0, 0):
            d.start()
            d.wait()
        eff_iters(0, 0, VECS_PER_MEGA)

        def mega_body(m, _):
            p = m % 2
            q = 1 - p
            m_next = jnp.minimum(m + 1, N_MEGA - 1)

            stage_descs = stage_start(q, m_next)
            for d in stage_descs:
                d.start()

            def gstart(j):
                d = pltpu.make_async_copy(
                    x_hbm.at[srcb.at[p, pl.ds(j * SUB, SUB)]],
                    rows.at[j % RB], gsem.at[j % RB])
                d.start()
                return d

            def estart(j):
                b = (ebase + m * MEGA + j * SUB) * D_EDGE
                d = pltpu.make_async_copy(
                    ea_hbm.at[pl.ds(b, SUB * D_EDGE)],
                    ea4.at[j % RB], wsem.at[j % RB])
                d.start()
                return d

            def widen(j):
                slot = jnp.full((16,), j % RB, jnp.int32)
                for i in range(SUB * D_EDGE // 16):
                    v = ea4[j % RB, pl.ds(i * 16, 16)]
                    plsc.store_scatter(ea16, [slot, i * 4 + row_off, col_off], v)

            gd = [None] * N_SUB
            ed = [None] * N_SUB
            sg = [None] * N_SUB
            se = [None] * N_SUB
            gd[0] = gstart(0)
            ed[0] = estart(0)
            for k in range(N_SUB):
                if k >= 1:
                    sg[k - 1].wait()
                    se[k - 1].wait()
                if k + 1 < N_SUB:
                    gd[k + 1] = gstart(k + 1)
                    ed[k + 1] = estart(k + 1)
                gd[k].wait()
                ed[k].wait()
                widen(k)
                idx = effb.at[p, k]
                sg[k] = pltpu.make_async_copy(rows.at[k % RB],
                                              g_sh.at[idx], sgsem.at[k % RB])
                sg[k].start(add=True)
                se[k] = pltpu.make_async_copy(ea16.at[k % RB],
                                              e_sh.at[idx], sesem.at[k % RB])
                se[k].start(add=True)
                if k == 0:
                    for d in stage_descs:
                        d.wait()
                elif k >= 2:
                    lo = (k - 2) * 9
                    hi = min(VECS_PER_MEGA, lo + 9)
                    eff_iters(q, lo, hi)
            sg[N_SUB - 1].wait()
            se[N_SUB - 1].wait()
            return 0

        lax.fori_loop(0, N_MEGA, mega_body, 0)

        plsc.subcore_barrier()

        pltpu.sync_copy(g_sh.at[pl.ds(r0, ROWS_PER_TILE)],
                        g_out.at[cid, pl.ds(r0, ROWS_PER_TILE)])
        pltpu.sync_copy(e_sh.at[pl.ds(r0, ROWS_PER_TILE)],
                        e_out.at[cid, pl.ds(r0, ROWS_PER_TILE)])

    return sc_kernel


def _dense_body(x_ref, gp_ref, ep_ref, ws_ref, wn_ref, we_ref, b_ref, m_ref, o_ref):
    g = gp_ref[0] + gp_ref[1]
    e = ep_ref[0] + ep_ref[1]
    acc = jnp.dot(x_ref[...], ws_ref[...], preferred_element_type=jnp.float32)
    acc = acc + jnp.dot(g, wn_ref[...], preferred_element_type=jnp.float32)
    acc = acc + jnp.dot(e, we_ref[...], preferred_element_type=jnp.float32)
    acc = acc + b_ref[...]
    o_ref[...] = jnp.maximum(acc, 0.0) * m_ref[...]


_R = 400


def _dense_call(x, gp, ep, W_self, W_nbr, W_edge, b2, m2):
    return pl.pallas_call(
        _dense_body,
        grid=(N_NODES // _R,),
        in_specs=[
            pl.BlockSpec((_R, D_FEAT), lambda i: (i, 0)),
            pl.BlockSpec((NC, _R, D_FEAT), lambda i: (0, i, 0)),
            pl.BlockSpec((NC, _R, D_EDGE_PAD), lambda i: (0, i, 0)),
            pl.BlockSpec((D_FEAT, D_FEAT), lambda i: (0, 0)),
            pl.BlockSpec((D_FEAT, D_FEAT), lambda i: (0, 0)),
            pl.BlockSpec((D_EDGE_PAD, D_FEAT), lambda i: (0, 0)),
            pl.BlockSpec((1, D_FEAT), lambda i: (0, 0)),
            pl.BlockSpec((_R, 1), lambda i: (i, 0)),
        ],
        out_specs=pl.BlockSpec((_R, D_FEAT), lambda i: (i, 0)),
        out_shape=jax.ShapeDtypeStruct((N_NODES, D_FEAT), jnp.float32),
    )(x, gp, ep, W_self, W_nbr, W_edge, b2, m2)


def kernel(x, edge_attr, W_self, W_nbr, W_edge, b, edge_index, mask):
    mask_i32 = mask.astype(jnp.int32)
    zg = jnp.zeros((ROWS_PAD, D_FEAT), jnp.float32)
    ze = jnp.zeros((ROWS_PAD, D_EDGE_PAD), jnp.float32)
    ea_flat = edge_attr.reshape(-1)
    sc = _make_sc_kernel()
    gp, ep = sc(x, edge_index, ea_flat, mask_i32, zg, ze)
    b2 = b.reshape(1, D_FEAT)
    m2 = mask.astype(jnp.float32).reshape(N_NODES, 1)
    we_pad = jnp.pad(W_edge, ((0, D_EDGE_PAD - D_EDGE), (0, 0)))
    return _dense_call(x, gp, ep, W_self, W_nbr, we_pad, b2, m2)


# speedup vs baseline: 2.7644x; 1.7303x over previous
import functools

import jax
import jax.numpy as jnp
from jax import lax
from jax.experimental import pallas as pl
from jax.experimental.pallas import tpu as pltpu
from jax.experimental.pallas import tpu_sc as plsc

N_NODES = 10000
N_EDGES = 320000
D_FEAT = 128
D_EDGE = 4
D_EDGE_PAD = 16

NC = 2
NS = 16
NW = NC * NS

ROWS_PAD = 10240
ROWS_PER_TILE = ROWS_PAD // NS
DUMMY_ROW = 10200
EDGES_PER_WORKER = N_EDGES // NW
MEGA = 400
N_MEGA = EDGES_PER_WORKER // MEGA
SUB = 80
N_SUB = MEGA // SUB
VECS_PER_MEGA = MEGA // 16
RB = 2


def _make_sc_kernel():
    mesh = plsc.VectorSubcoreMesh(core_axis_name="c", subcore_axis_name="s")

    @functools.partial(
        pl.kernel,
        out_type=[
            jax.ShapeDtypeStruct((NC, ROWS_PAD, D_FEAT), jnp.float32),
            jax.ShapeDtypeStruct((NC, ROWS_PAD, D_EDGE_PAD), jnp.float32),
        ],
        mesh=mesh,
        compiler_params=pltpu.CompilerParams(
            needs_layout_passes=False, use_tc_tiling_on_sc=False),
        scratch_types=[
            pltpu.VMEM((N_NODES,), jnp.int32),
            pltpu.VMEM((2, MEGA), jnp.int32),
            pltpu.VMEM((2, MEGA), jnp.int32),
            pltpu.VMEM((2, D_EDGE, MEGA), jnp.float32),
            pltpu.VMEM((RB, SUB, D_EDGE_PAD), jnp.float32),
            pltpu.VMEM((2, N_SUB, SUB), jnp.int32),
            pltpu.VMEM((RB, SUB, D_FEAT), jnp.float32),
            pltpu.VMEM_SHARED((ROWS_PAD, D_FEAT), jnp.float32),
            pltpu.VMEM_SHARED((ROWS_PAD, D_EDGE_PAD), jnp.float32),
            pltpu.SemaphoreType.DMA((3,)),
            pltpu.SemaphoreType.DMA((RB,)),
            pltpu.SemaphoreType.DMA((RB,)),
            pltpu.SemaphoreType.DMA((RB,)),
            pltpu.SemaphoreType.DMA((RB,)),
        ],
    )
    def sc_kernel(x_hbm, ei_hbm, ea_hbm, mask_hbm, zg_hbm, ze_hbm,
                  g_out, e_out,
                  mask_v, srcb, dstb, eacolb, ea16, effb, rows, g_sh, e_sh,
                  stsem, gsem, wsem, sgsem, sesem):
        cid = lax.axis_index("c")
        sid = lax.axis_index("s")
        wid = cid * NS + sid
        r0 = sid * ROWS_PER_TILE

        pltpu.sync_copy(zg_hbm.at[pl.ds(r0, ROWS_PER_TILE)],
                        g_sh.at[pl.ds(r0, ROWS_PER_TILE)])
        pltpu.sync_copy(ze_hbm.at[pl.ds(r0, ROWS_PER_TILE)],
                        e_sh.at[pl.ds(r0, ROWS_PER_TILE)])
        for j in range(RB):
            pltpu.sync_copy(ze_hbm.at[pl.ds(0, SUB)], ea16.at[j])
        pltpu.sync_copy(mask_hbm, mask_v)

        plsc.subcore_barrier()

        lane = lax.iota(jnp.int32, 16)

        ebase = wid * EDGES_PER_WORKER

        def stage_start(pp, mm):
            b = ebase + mm * MEGA
            return [
                pltpu.make_async_copy(ei_hbm.at[0, pl.ds(b, MEGA)],
                                      srcb.at[pp], stsem.at[0]),
                pltpu.make_async_copy(ei_hbm.at[1, pl.ds(b, MEGA)],
                                      dstb.at[pp], stsem.at[1]),
                pltpu.make_async_copy(ea_hbm.at[:, pl.ds(b, MEGA)],
                                      eacolb.at[pp], stsem.at[2]),
            ]

        def eff_iters(qq, lo, hi):
            def body(i, _):
                sv = srcb[qq, pl.ds(i * 16, 16)]
                dv = dstb[qq, pl.ds(i * 16, 16)]
                ms = plsc.load_gather(mask_v, [sv])
                md = plsc.load_gather(mask_v, [dv])
                keep = (ms & md) > 0
                effb[qq, i // 5, pl.ds((i % 5) * 16, 16)] = (
                    jnp.where(keep, dv, DUMMY_ROW))
                return 0
            lax.fori_loop(lo, hi, body, 0)

        for d in stage_start(0, 0):
            d.start()
            d.wait()
        eff_iters(0, 0, VECS_PER_MEGA)

        def mega_body(m, _):
            p = m % 2
            q = 1 - p
            m_next = jnp.minimum(m + 1, N_MEGA - 1)

            stage_descs = stage_start(q, m_next)
            for d in stage_descs:
                d.start()

            def gstart(j):
                d = pltpu.make_async_copy(
                    x_hbm.at[srcb.at[p, pl.ds(j * SUB, SUB)]],
                    rows.at[j % RB], gsem.at[j % RB])
                d.start()
                return d

            def widen(j, pp):
                slot = jnp.full((16,), j % RB, jnp.int32)
                for jc in range(D_EDGE):
                    colv = jnp.full((16,), jc, jnp.int32)
                    for i in range(SUB // 16):
                        v = eacolb[pp, jc, pl.ds(j * SUB + i * 16, 16)]
                        plsc.store_scatter(ea16, [slot, i * 16 + lane, colv], v)

            gd = [None] * N_SUB
            sg = [None] * N_SUB
            se = [None] * N_SUB
            gd[0] = gstart(0)
            for k in range(N_SUB):
                if k >= 1:
                    sg[k - 1].wait()
                    se[k - 1].wait()
                if k + 1 < N_SUB:
                    gd[k + 1] = gstart(k + 1)
                widen(k, p)
                gd[k].wait()
                idx = effb.at[p, k]
                sg[k] = pltpu.make_async_copy(rows.at[k % RB],
                                              g_sh.at[idx], sgsem.at[k % RB])
                sg[k].start(add=True)
                se[k] = pltpu.make_async_copy(ea16.at[k % RB],
                                              e_sh.at[idx], sesem.at[k % RB])
                se[k].start(add=True)
                if k == 0:
                    for d in stage_descs:
                        d.wait()
                elif k >= 2:
                    lo = (k - 2) * 9
                    hi = min(VECS_PER_MEGA, lo + 9)
                    eff_iters(q, lo, hi)
            sg[N_SUB - 1].wait()
            se[N_SUB - 1].wait()
            return 0

        lax.fori_loop(0, N_MEGA, mega_body, 0)

        plsc.subcore_barrier()

        pltpu.sync_copy(g_sh.at[pl.ds(r0, ROWS_PER_TILE)],
                        g_out.at[cid, pl.ds(r0, ROWS_PER_TILE)])
        pltpu.sync_copy(e_sh.at[pl.ds(r0, ROWS_PER_TILE)],
                        e_out.at[cid, pl.ds(r0, ROWS_PER_TILE)])

    return sc_kernel


def _dense_body(x_ref, gp_ref, ep_ref, ws_ref, wn_ref, we_ref, b_ref, m_ref, o_ref):
    g = gp_ref[0] + gp_ref[1]
    e = ep_ref[0] + ep_ref[1]
    acc = jnp.dot(x_ref[...], ws_ref[...], preferred_element_type=jnp.float32)
    acc = acc + jnp.dot(g, wn_ref[...], preferred_element_type=jnp.float32)
    acc = acc + jnp.dot(e, we_ref[...], preferred_element_type=jnp.float32)
    acc = acc + b_ref[...]
    o_ref[...] = jnp.maximum(acc, 0.0) * m_ref[...]


_R = 400


def _dense_call(x, gp, ep, W_self, W_nbr, W_edge, b2, m2):
    return pl.pallas_call(
        _dense_body,
        grid=(N_NODES // _R,),
        in_specs=[
            pl.BlockSpec((_R, D_FEAT), lambda i: (i, 0)),
            pl.BlockSpec((NC, _R, D_FEAT), lambda i: (0, i, 0)),
            pl.BlockSpec((NC, _R, D_EDGE_PAD), lambda i: (0, i, 0)),
            pl.BlockSpec((D_FEAT, D_FEAT), lambda i: (0, 0)),
            pl.BlockSpec((D_FEAT, D_FEAT), lambda i: (0, 0)),
            pl.BlockSpec((D_EDGE_PAD, D_FEAT), lambda i: (0, 0)),
            pl.BlockSpec((1, D_FEAT), lambda i: (0, 0)),
            pl.BlockSpec((_R, 1), lambda i: (i, 0)),
        ],
        out_specs=pl.BlockSpec((_R, D_FEAT), lambda i: (i, 0)),
        out_shape=jax.ShapeDtypeStruct((N_NODES, D_FEAT), jnp.float32),
    )(x, gp, ep, W_self, W_nbr, W_edge, b2, m2)


def kernel(x, edge_attr, W_self, W_nbr, W_edge, b, edge_index, mask):
    mask_i32 = mask.astype(jnp.int32)
    zg = jnp.zeros((ROWS_PAD, D_FEAT), jnp.float32)
    ze = jnp.zeros((ROWS_PAD, D_EDGE_PAD), jnp.float32)
    ea_t = edge_attr.T
    sc = _make_sc_kernel()
    gp, ep = sc(x, edge_index, ea_t, mask_i32, zg, ze)
    b2 = b.reshape(1, D_FEAT)
    m2 = mask.astype(jnp.float32).reshape(N_NODES, 1)
    we_pad = jnp.pad(W_edge, ((0, D_EDGE_PAD - D_EDGE), (0, 0)))
    return _dense_call(x, gp, ep, W_self, W_nbr, we_pad, b2, m2)
